# baseline (speedup 1.0000x reference)
import functools

import jax
import jax.numpy as jnp
import numpy as np
from jax import lax
from jax.experimental import pallas as pl
from jax.experimental.pallas import tpu as pltpu
from jax.experimental.pallas import tpu_sc as plsc

F = 128
D = 128
K = 16
LANES = 16
NSUB = F // LANES
NEG_SLOPE = np.float32(0.2)
GW = 2


@functools.lru_cache(maxsize=None)
def _make_sc_attend(B: int, C: int, NBUF: int):
  info = plsc.get_sparse_core_info()
  nw = info.num_cores * info.num_subcores
  rpw = B // nw
  nch = rpw // C
  assert nch % NBUF == 0
  mesh = plsc.VectorSubcoreMesh(core_axis_name="c", subcore_axis_name="s")

  @functools.partial(
      pl.kernel,
      out_type=[
          jax.ShapeDtypeStruct((B, F), jnp.float32),
          jax.ShapeDtypeStruct((B, F), jnp.float32),
          jax.ShapeDtypeStruct((B, K), jnp.float32),
      ],
      mesh=mesh,
      compiler_params=pltpu.CompilerParams(needs_layout_passes=False),
      scratch_types=[
          pltpu.VMEM((rpw,), jnp.int32),
          pltpu.VMEM((NBUF, C * K), jnp.int32),
          pltpu.VMEM((NBUF, C * K), jnp.int32),
          pltpu.VMEM((NBUF, C, F), jnp.float32),
          pltpu.VMEM((NBUF, C * K, F), jnp.float32),
          pltpu.VMEM((NBUF, C, F), jnp.float32),
          pltpu.VMEM((NBUF, C, K), jnp.float32),
          pltpu.VMEM((NBUF, C), jnp.float32),
          pltpu.VMEM((NBUF, C * K), jnp.float32),
          pltpu.VMEM_SHARED((2 * 100352,), jnp.float32),
          pltpu.VMEM((100352 // 8,), jnp.float32),
      ] + [pltpu.SemaphoreType.DMA] * (6 * NBUF),
  )
  def sc_attend(table, g, nodes1d, nidx1d,
                emb2_o, s_o, ev_o,
                nodes_all, idxb, gidxf,
                e2b, nbb, sb, evb, gnb, gxb, g_sh, gtmp, *sems):
    semg1 = sems[0:NBUF]
    semg2 = sems[NBUF:2 * NBUF]
    semw = sems[2 * NBUF:3 * NBUF]
    semg3 = sems[3 * NBUF:4 * NBUF]
    semg4 = sems[4 * NBUF:5 * NBUF]
    semi = sems[5 * NBUF:6 * NBUF]
    sid = lax.axis_index("s")
    wid = sid * info.num_cores + lax.axis_index("c")
    base0 = wid * rpw
    npad = g.shape[1]
    stripe = gtmp.shape[0]
    srow = sid // 8
    scol = pl.multiple_of((sid % 8) * stripe, 8)
    pltpu.sync_copy(g.at[srow, pl.ds(scol, stripe)], gtmp)
    pltpu.sync_copy(
        gtmp, g_sh.at[pl.ds(pl.multiple_of(srow * npad + scol, 8), stripe)])
    pltpu.sync_copy(nodes1d.at[pl.ds(base0, rpw)], nodes_all)
    plsc.subcore_barrier()

    def idx_cp(i, b):
      return pltpu.make_async_copy(
          nidx1d.at[pl.ds((base0 + i * C) * K, C * K)], idxb.at[b], semi[b])

    def repack(b):
      for r in range(C):
        v = idxb[b, pl.ds(r * K, K)]
        gidxf[b, pl.ds(r * K, K)] = v + npad

    def gather_cps(i, b):
      return (pltpu.make_async_copy(table.at[nodes_all.at[pl.ds(i * C, C)]],
                                    e2b.at[b], semg1[b]),
              pltpu.make_async_copy(g_sh.at[nodes_all.at[pl.ds(i * C, C)]],
                                    gnb.at[b], semg3[b]),
              pltpu.make_async_copy(table.at[idxb.at[b]], nbb.at[b],
                                    semg2[b]),
              pltpu.make_async_copy(g_sh.at[gidxf.at[b]], gxb.at[b],
                                    semg4[b]))

    def wb_cps(i, b):
      base = base0 + i * C
      return (pltpu.make_async_copy(e2b.at[b], emb2_o.at[pl.ds(base, C)],
                                    semw[b]),
              pltpu.make_async_copy(sb.at[b], s_o.at[pl.ds(base, C)],
                                    semw[b]),
              pltpu.make_async_copy(evb.at[b], ev_o.at[pl.ds(base, C)],
                                    semw[b]))

    def compute(b):
      def row_body(r, rcarry):
        gv = gxb[b, pl.ds(r * K, K)]
        c2 = plsc.load_gather(gnb.at[b], [jnp.full((LANES,), r, jnp.int32)])
        lg = gv + c2
        lr = jnp.where(lg >= 0, lg, lg * NEG_SLOPE)
        ev = jnp.exp(-lr)
        evb[b, r, :] = ev
        acc = [None] * NSUB
        for k in range(K):
          ek = plsc.load_gather(evb.at[b],
                                [jnp.full((LANES,), r, jnp.int32),
                                 jnp.full((LANES,), k, jnp.int32)])
          nv = [nbb[b, r * K + k, pl.ds(LANES * j, LANES)]
                for j in range(NSUB)]
          for j in range(NSUB):
            acc[j] = ek * nv[j] if k == 0 else acc[j] + ek * nv[j]
        for j in range(NSUB):
          sb[b, r, pl.ds(LANES * j, LANES)] = acc[j]
        return rcarry

      lax.fori_loop(0, C, row_body, 0)

    for b in range(NBUF):
      idx_cp(b, b).start()
    for b in range(NBUF):
      idx_cp(b, b).wait()
      repack(b)
      for cp in gather_cps(b, b):
        cp.start()

    def group(gi, carry):
      for b in range(NBUF):
        i = gi * NBUF + b
        for cp in gather_cps(i, b):
          cp.wait()
        inext = i + NBUF

        @pl.when(inext < nch)
        def _prefetch_idx():
          idx_cp(inext, b).start()

        compute(b)
        for cp in wb_cps(i, b):
          cp.start()

        @pl.when(inext < nch)
        def _reissue():
          idx_cp(inext, b).wait()
          repack(b)
          for cp in wb_cps(i, b):
            cp.wait()
          for cp in gather_cps(inext, b):
            cp.start()

      return carry

    lax.fori_loop(0, nch // NBUF, group, 0)
    for b in range(NBUF):
      for cp in wb_cps(nch - NBUF + b, b):
        cp.wait()

  return sc_attend


def _tc_project(table, w12):
  n = table.shape[0]
  bn = 12544
  nblk = -(-n // bn)
  npad = nblk * bn

  def body(w_r, t_r, o_r):
    o_r[...] = lax.dot_general(w_r[...], t_r[...], (((1,), (1,)), ((), ())),
                               preferred_element_type=jnp.float32)

  return pl.pallas_call(
      body,
      grid=(nblk,),
      in_specs=[
          pl.BlockSpec((2, F), lambda i: (0, 0)),
          pl.BlockSpec((bn, F), lambda i: (i, 0)),
      ],
      out_specs=pl.BlockSpec((2, bn), lambda i: (0, i)),
      out_shape=jax.ShapeDtypeStruct((2, npad), jnp.float32),
  )(w12, table)


def _tc_finish(emb2, s, ev, nf, W, wd1, wd2, wd3):
  B = emb2.shape[0]
  BM = 4096

  def body(e2_r, s_r, ev_r, nf_r, w_r, wd1_r, wd2_r, wd3_r, o_r):
    rs = jnp.sum(ev_r[...], axis=1, keepdims=True)
    hp = jnp.dot(s_r[...], w_r[...], preferred_element_type=jnp.float32)
    hp = jnp.nan_to_num(hp)
    hp = jnp.nan_to_num(hp / rs)
    acc = jnp.dot(e2_r[...], wd1_r[...], preferred_element_type=jnp.float32)
    acc = acc + jnp.dot(hp, wd2_r[...], preferred_element_type=jnp.float32)
    acc = acc + jnp.dot(nf_r[...], wd3_r[...], preferred_element_type=jnp.float32)
    o_r[...] = jnp.maximum(acc, 0.0)

  return pl.pallas_call(
      body,
      grid=(B // BM,),
      in_specs=[
          pl.BlockSpec((BM, F), lambda i: (i, 0)),
          pl.BlockSpec((BM, F), lambda i: (i, 0)),
          pl.BlockSpec((BM, K), lambda i: (i, 0)),
          pl.BlockSpec((BM, F), lambda i: (i, 0)),
          pl.BlockSpec((F, D), lambda i: (0, 0)),
          pl.BlockSpec((F, D), lambda i: (0, 0)),
          pl.BlockSpec((D, D), lambda i: (0, 0)),
          pl.BlockSpec((F, D), lambda i: (0, 0)),
      ],
      out_specs=pl.BlockSpec((BM, D), lambda i: (i, 0)),
      out_shape=jax.ShapeDtypeStruct((B, D), jnp.float32),
  )(emb2, s, ev, nf, W, wd1, wd2, wd3)


@jax.jit
def kernel(nodes, neigh_idx, neigh_feats, feat_table, W, a_param,
           detaching_weight):
  B, _ = neigh_idx.shape
  C, NBUF = 8, 4
  nodes1d = nodes.astype(jnp.int32)
  nidx32 = neigh_idx.reshape(-1).astype(jnp.int32)
  w12 = a_param.reshape(2, D).astype(jnp.float32) @ W.T
  g = _tc_project(feat_table, w12)
  sc = _make_sc_attend(B, C, NBUF)
  emb2, s, ev = sc(feat_table, g, nodes1d, nidx32)
  wd1 = detaching_weight[:F]
  wd2 = detaching_weight[F:F + D]
  wd3 = detaching_weight[F + D:]
  return _tc_finish(emb2, s, ev, neigh_feats, W, wd1, wd2, wd3)


# speedup vs baseline: 1.0412x; 1.0412x over previous
import functools

import jax
import jax.numpy as jnp
import numpy as np
from jax import lax
from jax.experimental import pallas as pl
from jax.experimental.pallas import tpu as pltpu
from jax.experimental.pallas import tpu_sc as plsc

F = 128
D = 128
K = 16
LANES = 16
NSUB = F // LANES
NEG_SLOPE = np.float32(0.2)
GW = 2


@functools.lru_cache(maxsize=None)
def _make_sc_attend(B: int, C: int, NBUF: int):
  info = plsc.get_sparse_core_info()
  nw = info.num_cores * info.num_subcores
  rpw = B // nw
  nch = rpw // C
  assert nch % NBUF == 0
  mesh = plsc.VectorSubcoreMesh(core_axis_name="c", subcore_axis_name="s")

  @functools.partial(
      pl.kernel,
      out_type=[
          jax.ShapeDtypeStruct((B, F), jnp.float32),
          jax.ShapeDtypeStruct((B, F), jnp.float32),
          jax.ShapeDtypeStruct((B, K), jnp.float32),
      ],
      mesh=mesh,
      compiler_params=pltpu.CompilerParams(needs_layout_passes=False),
      scratch_types=[
          pltpu.VMEM((rpw,), jnp.int32),
          pltpu.VMEM((NBUF, C, K), jnp.int32),
          pltpu.VMEM((NBUF, C * K), jnp.int32),
          pltpu.VMEM((NBUF, C * K), jnp.int32),
          pltpu.VMEM((NBUF, C, F), jnp.float32),
          pltpu.VMEM((NBUF, C * K, F), jnp.float32),
          pltpu.VMEM((NBUF, C, F), jnp.float32),
          pltpu.VMEM((NBUF, C, K), jnp.float32),
          pltpu.VMEM((NBUF, C), jnp.float32),
          pltpu.VMEM((NBUF, C * K), jnp.float32),
          pltpu.VMEM_SHARED((2 * 100352,), jnp.float32),
          pltpu.VMEM((100352 // 8,), jnp.float32),
      ] + [pltpu.SemaphoreType.DMA] * (6 * NBUF),
  )
  def sc_attend(table, g, nodes1d, nidx2d,
                emb2_o, s_o, ev_o,
                nodes_all, idxb, idxf, gidxf,
                e2b, nbb, sb, evb, gnb, gxb, g_sh, gtmp, *sems):
    semg1 = sems[0:NBUF]
    semg2 = sems[NBUF:2 * NBUF]
    semw = sems[2 * NBUF:3 * NBUF]
    semg3 = sems[3 * NBUF:4 * NBUF]
    semg4 = sems[4 * NBUF:5 * NBUF]
    semi = sems[5 * NBUF:6 * NBUF]
    sid = lax.axis_index("s")
    wid = sid * info.num_cores + lax.axis_index("c")
    base0 = wid * rpw
    npad = g.shape[1]
    stripe = gtmp.shape[0]
    srow = sid // 8
    scol = pl.multiple_of((sid % 8) * stripe, 8)
    pltpu.sync_copy(g.at[srow, pl.ds(scol, stripe)], gtmp)
    pltpu.sync_copy(
        gtmp, g_sh.at[pl.ds(pl.multiple_of(srow * npad + scol, 8), stripe)])
    pltpu.sync_copy(nodes1d.at[pl.ds(base0, rpw)], nodes_all)
    plsc.subcore_barrier()

    def idx_cp(i, b):
      return pltpu.make_async_copy(nidx2d.at[pl.ds(base0 + i * C, C)],
                                   idxb.at[b], semi[b])

    def repack(b):
      for r in range(C):
        v = idxb[b, r, :]
        idxf[b, pl.ds(r * K, K)] = v
        gidxf[b, pl.ds(r * K, K)] = v + npad

    def gather_cps(i, b):
      return (pltpu.make_async_copy(table.at[nodes_all.at[pl.ds(i * C, C)]],
                                    e2b.at[b], semg1[b]),
              pltpu.make_async_copy(g_sh.at[nodes_all.at[pl.ds(i * C, C)]],
                                    gnb.at[b], semg3[b]),
              pltpu.make_async_copy(table.at[idxf.at[b]], nbb.at[b],
                                    semg2[b]),
              pltpu.make_async_copy(g_sh.at[gidxf.at[b]], gxb.at[b],
                                    semg4[b]))

    def wb_cps(i, b):
      base = base0 + i * C
      return (pltpu.make_async_copy(e2b.at[b], emb2_o.at[pl.ds(base, C)],
                                    semw[b]),
              pltpu.make_async_copy(sb.at[b], s_o.at[pl.ds(base, C)],
                                    semw[b]),
              pltpu.make_async_copy(evb.at[b], ev_o.at[pl.ds(base, C)],
                                    semw[b]))

    def compute(b):
      def row_body(r, rcarry):
        gv = gxb[b, pl.ds(r * K, K)]
        c2 = plsc.load_gather(gnb.at[b], [jnp.full((LANES,), r, jnp.int32)])
        lg = gv + c2
        lr = jnp.where(lg >= 0, lg, lg * NEG_SLOPE)
        ev = jnp.exp(-lr)
        evb[b, r, :] = ev
        acc = [None] * NSUB
        for k in range(K):
          ek = plsc.load_gather(evb.at[b],
                                [jnp.full((LANES,), r, jnp.int32),
                                 jnp.full((LANES,), k, jnp.int32)])
          nv = [nbb[b, r * K + k, pl.ds(LANES * j, LANES)]
                for j in range(NSUB)]
          for j in range(NSUB):
            acc[j] = ek * nv[j] if k == 0 else acc[j] + ek * nv[j]
        for j in range(NSUB):
          sb[b, r, pl.ds(LANES * j, LANES)] = acc[j]
        return rcarry

      lax.fori_loop(0, C, row_body, 0)

    for b in range(NBUF):
      idx_cp(b, b).start()
    for b in range(NBUF):
      idx_cp(b, b).wait()
      repack(b)
      for cp in gather_cps(b, b):
        cp.start()

    def group(gi, carry):
      for b in range(NBUF):
        i = gi * NBUF + b
        for cp in gather_cps(i, b):
          cp.wait()
        inext = i + NBUF

        @pl.when(inext < nch)
        def _prefetch_idx():
          idx_cp(inext, b).start()

        compute(b)
        for cp in wb_cps(i, b):
          cp.start()

        @pl.when(inext < nch)
        def _reissue():
          idx_cp(inext, b).wait()
          repack(b)
          for cp in wb_cps(i, b):
            cp.wait()
          for cp in gather_cps(inext, b):
            cp.start()

      return carry

    lax.fori_loop(0, nch // NBUF, group, 0)
    for b in range(NBUF):
      for cp in wb_cps(nch - NBUF + b, b):
        cp.wait()

  return sc_attend


def _tc_project(table, w12):
  n = table.shape[0]
  bn = 12544
  nblk = -(-n // bn)
  npad = nblk * bn

  def body(w_r, t_r, o_r):
    o_r[...] = lax.dot_general(w_r[...], t_r[...], (((1,), (1,)), ((), ())),
                               preferred_element_type=jnp.float32)

  return pl.pallas_call(
      body,
      grid=(nblk,),
      in_specs=[
          pl.BlockSpec((2, F), lambda i: (0, 0)),
          pl.BlockSpec((bn, F), lambda i: (i, 0)),
      ],
      out_specs=pl.BlockSpec((2, bn), lambda i: (0, i)),
      out_shape=jax.ShapeDtypeStruct((2, npad), jnp.float32),
  )(w12, table)


def _tc_finish(emb2, s, ev, nf, W, wd1, wd2, wd3):
  B = emb2.shape[0]
  BM = 4096

  def body(e2_r, s_r, ev_r, nf_r, w_r, wd1_r, wd2_r, wd3_r, o_r):
    rs = jnp.sum(ev_r[...], axis=1, keepdims=True)
    hp = jnp.dot(s_r[...], w_r[...], preferred_element_type=jnp.float32)
    hp = jnp.nan_to_num(hp)
    hp = jnp.nan_to_num(hp / rs)
    acc = jnp.dot(e2_r[...], wd1_r[...], preferred_element_type=jnp.float32)
    acc = acc + jnp.dot(hp, wd2_r[...], preferred_element_type=jnp.float32)
    acc = acc + jnp.dot(nf_r[...], wd3_r[...], preferred_element_type=jnp.float32)
    o_r[...] = jnp.maximum(acc, 0.0)

  return pl.pallas_call(
      body,
      grid=(B // BM,),
      in_specs=[
          pl.BlockSpec((BM, F), lambda i: (i, 0)),
          pl.BlockSpec((BM, F), lambda i: (i, 0)),
          pl.BlockSpec((BM, K), lambda i: (i, 0)),
          pl.BlockSpec((BM, F), lambda i: (i, 0)),
          pl.BlockSpec((F, D), lambda i: (0, 0)),
          pl.BlockSpec((F, D), lambda i: (0, 0)),
          pl.BlockSpec((D, D), lambda i: (0, 0)),
          pl.BlockSpec((F, D), lambda i: (0, 0)),
      ],
      out_specs=pl.BlockSpec((BM, D), lambda i: (i, 0)),
      out_shape=jax.ShapeDtypeStruct((B, D), jnp.float32),
  )(emb2, s, ev, nf, W, wd1, wd2, wd3)


@jax.jit
def kernel(nodes, neigh_idx, neigh_feats, feat_table, W, a_param,
           detaching_weight):
  B, _ = neigh_idx.shape
  C, NBUF = 8, 4
  nodes1d = nodes.astype(jnp.int32)
  nidx32 = neigh_idx.astype(jnp.int32)
  w12 = a_param.reshape(2, D).astype(jnp.float32) @ W.T
  g = _tc_project(feat_table, w12)
  sc = _make_sc_attend(B, C, NBUF)
  emb2, s, ev = sc(feat_table, g, nodes1d, nidx32)
  wd1 = detaching_weight[:F]
  wd2 = detaching_weight[F:F + D]
  wd3 = detaching_weight[F + D:]
  return _tc_finish(emb2, s, ev, neigh_feats, W, wd1, wd2, wd3)
